# baseline (speedup 1.0000x reference)
import functools

import jax
import jax.numpy as jnp
from jax import lax
from jax.experimental import pallas as pl
from jax.experimental.pallas import tpu as pltpu
from jax.experimental.pallas import tpu_sc as plsc

N = 10000
E = 320000
D = 128
H = 256
C = 64
K_PROP = 10
ALPHA = 0.1

NC = 2
NS = 16
NW = NC * NS
EPW = E // NW
CE = 125
NB = EPW // CE

HC = C // NC
TPE = E // NS
GB = 125
NGB = TPE // GB
NBUF = 8
LA = 4
RPT = N // NS
RCH = 125
NCH = RPT // RCH

_mesh = plsc.VectorSubcoreMesh(core_axis_name="c", subcore_axis_name="s",
                               num_cores=NC, num_subcores=NS)
_sc_params = pltpu.CompilerParams(use_tc_tiling_on_sc=False)


def _worker_id():
    return lax.axis_index("s") * NC + lax.axis_index("c")


@functools.partial(
    pl.kernel,
    out_type=jax.ShapeDtypeStruct((NC, 2, N), jnp.float32),
    mesh=_mesh,
    compiler_params=_sc_params,
    scratch_types=[
        pltpu.VMEM((NB, CE), jnp.int32),
        pltpu.VMEM((NB, CE), jnp.int32),
        pltpu.VMEM((128,), jnp.float32),
        pltpu.VMEM((2000,), jnp.float32),
        pltpu.VMEM_SHARED((N,), jnp.float32),
        pltpu.VMEM_SHARED((N,), jnp.float32),
    ],
)
def _deg_kernel(src_hbm, dst_hbm, dpart, src_v, dst_v, ones_v, z_v,
                acc_out, acc_in):
    cid = lax.axis_index("c")
    sid = lax.axis_index("s")
    wid = _worker_id()

    def fill_ones(i, _):
        ones_v[pl.ds(i * 16, 16)] = jnp.ones((16,), jnp.float32)
        return 0
    lax.fori_loop(0, 128 // 16, fill_ones, 0)

    def fill_z(i, _):
        z_v[pl.ds(i * 16, 16)] = jnp.zeros((16,), jnp.float32)
        return 0
    lax.fori_loop(0, 2000 // 16, fill_z, 0)

    @pl.when(sid < 5)
    def _():
        pltpu.sync_copy(z_v, acc_out.at[pl.ds(sid * 2000, 2000)])
        pltpu.sync_copy(z_v, acc_in.at[pl.ds(sid * 2000, 2000)])

    plsc.subcore_barrier()

    pltpu.sync_copy(src_hbm.at[wid], src_v)
    pltpu.sync_copy(dst_hbm.at[wid], dst_v)

    def body(j, _):
        pltpu.sync_copy(ones_v.at[pl.ds(0, CE)], acc_out.at[src_v.at[j]],
                        add=True)
        pltpu.sync_copy(ones_v.at[pl.ds(0, CE)], acc_in.at[dst_v.at[j]],
                        add=True)
        return 0
    lax.fori_loop(0, NB, body, 0)

    plsc.subcore_barrier()

    @pl.when(sid < 10)
    def _():
        pltpu.sync_copy(acc_out.at[pl.ds(sid * 1000, 1000)],
                        dpart.at[cid, 0, pl.ds(sid * 1000, 1000)])
        pltpu.sync_copy(acc_in.at[pl.ds(sid * 1000, 1000)],
                        dpart.at[cid, 1, pl.ds(sid * 1000, 1000)])


_BR = 1000


def _mlp_body(feat, w0, b0, w1, b1, w2, b2, dpo, dpi,
              s0_ref, h0a_ref, nin_ref, nout_ref):
    x = feat[...]
    h = jnp.maximum(jnp.dot(x, w0[...], preferred_element_type=jnp.float32)
                    + b0[...], 0.0)
    h = jnp.maximum(jnp.dot(h, w1[...], preferred_element_type=jnp.float32)
                    + b1[...], 0.0)
    h = jnp.dot(h, w2[...], preferred_element_type=jnp.float32) + b2[...]

    dout = jnp.maximum(dpo[0] + dpo[1], 1.0)
    din = jnp.maximum(dpi[0] + dpi[1], 1.0)
    no = lax.rsqrt(dout)
    ni = lax.rsqrt(din)

    s0_ref[...] = h * no
    h0a_ref[...] = ALPHA * h
    nin_ref[...] = (1.0 - ALPHA) * ni
    nout_ref[...] = no


def _mlp_kernel(features, W0, b0, W1, b1, W2, b2, dpo, dpi):
    grid = (N // _BR,)
    outm = jax.ShapeDtypeStruct((N, C), jnp.float32)
    outv = jax.ShapeDtypeStruct((N, 1), jnp.float32)
    return pl.pallas_call(
        _mlp_body,
        grid=grid,
        in_specs=[
            pl.BlockSpec((_BR, D), lambda i: (i, 0)),
            pl.BlockSpec((D, H), lambda i: (0, 0)),
            pl.BlockSpec((1, H), lambda i: (0, 0)),
            pl.BlockSpec((H, H), lambda i: (0, 0)),
            pl.BlockSpec((1, H), lambda i: (0, 0)),
            pl.BlockSpec((H, C), lambda i: (0, 0)),
            pl.BlockSpec((1, C), lambda i: (0, 0)),
            pl.BlockSpec((NC, _BR, 1), lambda i: (0, i, 0)),
            pl.BlockSpec((NC, _BR, 1), lambda i: (0, i, 0)),
        ],
        out_specs=[
            pl.BlockSpec((_BR, C), lambda i: (i, 0)),
            pl.BlockSpec((_BR, C), lambda i: (i, 0)),
            pl.BlockSpec((_BR, 1), lambda i: (i, 0)),
            pl.BlockSpec((_BR, 1), lambda i: (i, 0)),
        ],
        out_shape=[outm, outm, outv, outv],
    )(features, W0, b0.reshape(1, H), W1, b1.reshape(1, H),
      W2, b2.reshape(1, C), dpo, dpi)


@functools.partial(
    pl.kernel,
    out_type=jax.ShapeDtypeStruct((NC * N, HC), jnp.float32),
    mesh=_mesh,
    compiler_params=_sc_params,
    scratch_types=[
        pltpu.VMEM((NGB, GB), jnp.int32),
        pltpu.VMEM((NGB, GB), jnp.int32),
        pltpu.VMEM((NBUF, GB, HC), jnp.float32),
        pltpu.VMEM((RCH, HC), jnp.float32),
        pltpu.VMEM_SHARED((N, HC), jnp.float32),
        pltpu.VMEM_SHARED((N, HC), jnp.float32),
        pltpu.SemaphoreType.DMA((NBUF,)),
        pltpu.SemaphoreType.DMA((NBUF,)),
    ],
)
def _prop_kernel(s_init, h0a2, ninm, noutm, srcr, dstr, s_buf,
                 src_v, dst_v, rows_v, zch, acc, stab, gsem, ssem):
    accv = rows_v.at[4]
    h0av = rows_v.at[5]
    ninv = rows_v.at[6]
    noutv = rows_v.at[7]
    outv = rows_v.at[4]
    cid = lax.axis_index("c")
    sid = lax.axis_index("s")
    base_row = sid * RPT

    off = cid * N

    pltpu.sync_copy(srcr.at[sid], src_v)
    pltpu.sync_copy(dstr.at[sid], dst_v)

    def fz(i, _):
        zch[i // (HC // 16), pl.ds((i % (HC // 16)) * 16, 16)] = (
            jnp.zeros((16,), jnp.float32))
        return 0
    lax.fori_loop(0, RCH * (HC // 16), fz, 0)

    def cinit(t, _):
        r0 = base_row + t * RCH
        pltpu.sync_copy(s_init.at[pl.ds(off + r0, RCH)], accv)
        pltpu.sync_copy(accv, stab.at[pl.ds(r0, RCH)])
        pltpu.sync_copy(zch, acc.at[pl.ds(r0, RCH)])
        return 0
    lax.fori_loop(0, NCH, cinit, 0)

    plsc.subcore_barrier()

    def step(k, _):
        for p in range(LA):
            pltpu.async_copy(stab.at[src_v.at[p]], rows_v.at[p],
                             gsem.at[p])

        def body(j, _):
            b = j % NBUF
            ahead = (j + LA) % NBUF

            @pl.when(j >= NBUF - LA)
            def _():
                pltpu.make_async_copy(rows_v.at[ahead],
                                      acc.at[dst_v.at[j - (NBUF - LA)]],
                                      ssem.at[ahead]).wait()

            @pl.when(j + LA < NGB)
            def _():
                pltpu.async_copy(stab.at[src_v.at[j + LA]],
                                 rows_v.at[ahead], gsem.at[ahead])

            pltpu.make_async_copy(stab.at[src_v.at[j]], rows_v.at[b],
                                  gsem.at[b]).wait()
            pltpu.async_copy(rows_v.at[b], acc.at[dst_v.at[j]], ssem.at[b],
                             add=True)
            return 0
        lax.fori_loop(0, NGB, body, 0)

        def drain(d, _):
            j = NGB - (NBUF - LA) + d
            pltpu.make_async_copy(rows_v.at[j % NBUF],
                                  acc.at[dst_v.at[j]],
                                  ssem.at[j % NBUF]).wait()
            return 0
        lax.fori_loop(0, NBUF - LA, drain, 0)

        plsc.subcore_barrier()

        last = k == K_PROP - 1

        def comb(t, _):
            r0 = base_row + t * RCH
            pltpu.sync_copy(acc.at[pl.ds(r0, RCH)], accv)
            pltpu.sync_copy(zch, acc.at[pl.ds(r0, RCH)])
            pltpu.sync_copy(h0a2.at[pl.ds(off + r0, RCH)], h0av)
            pltpu.sync_copy(ninm.at[pl.ds(off + r0, RCH)], ninv)
            pltpu.sync_copy(noutm.at[pl.ds(off + r0, RCH)], noutv)

            def rowloop(i, _):
                r = i // (HC // 16)
                sl = pl.ds((i % (HC // 16)) * 16, 16)
                no = jnp.where(last, 1.0, noutv[r, sl])
                outv[r, sl] = (accv[r, sl] * ninv[r, sl]
                               + h0av[r, sl]) * no
                return 0
            lax.fori_loop(0, RCH * (HC // 16), rowloop, 0)

            pltpu.sync_copy(outv, stab.at[pl.ds(r0, RCH)])

            @pl.when(last)
            def _():
                pltpu.sync_copy(outv, s_buf.at[pl.ds(off + r0, RCH)])
            return 0
        lax.fori_loop(0, NCH, comb, 0)

        plsc.subcore_barrier()
        return 0
    lax.fori_loop(0, K_PROP, step, 0)


def _split_cols(x):
    return x.reshape(N, NC, HC).transpose(1, 0, 2).reshape(NC * N, HC)


def kernel(features, edge_index, W0, b0, W1, b1, W2, b2):
    src = edge_index[0]
    dst = edge_index[1]

    dpart = _deg_kernel(src.reshape(NW, NB, CE), dst.reshape(NW, NB, CE))
    dpo = dpart[:, 0, :].reshape(NC, N, 1)
    dpi = dpart[:, 1, :].reshape(NC, N, 1)

    s0, h0a, nin1, nout = _mlp_kernel(features, W0, b0, W1, b1, W2, b2,
                                      dpo, dpi)

    ninm = _split_cols(jnp.broadcast_to(nin1, (N, C)))
    noutm = _split_cols(jnp.broadcast_to(nout, (N, C)))
    s_buf = _prop_kernel(_split_cols(s0), _split_cols(h0a), ninm, noutm,
                         src.reshape(NS, NGB, GB), dst.reshape(NS, NGB, GB))

    return s_buf.reshape(NC, N, HC).transpose(1, 0, 2).reshape(N, C)


# speedup vs baseline: 1.0748x; 1.0748x over previous
import functools

import jax
import jax.numpy as jnp
from jax import lax
from jax.experimental import pallas as pl
from jax.experimental.pallas import tpu as pltpu
from jax.experimental.pallas import tpu_sc as plsc

N = 10000
E = 320000
D = 128
H = 256
C = 64
K_PROP = 10
ALPHA = 0.1

NC = 2
NS = 16
NW = NC * NS
EPW = E // NW
CE = 125
NB = EPW // CE

HC = C // NC
TPE = E // NS
GB = 125
NGB = TPE // GB
NBUF = 8
LA = 4
RPT = N // NS
RCH = 125
NCH = RPT // RCH

_mesh = plsc.VectorSubcoreMesh(core_axis_name="c", subcore_axis_name="s",
                               num_cores=NC, num_subcores=NS)
_sc_params = pltpu.CompilerParams(use_tc_tiling_on_sc=False)


def _worker_id():
    return lax.axis_index("s") * NC + lax.axis_index("c")


@functools.partial(
    pl.kernel,
    out_type=jax.ShapeDtypeStruct((NC, 2, N), jnp.float32),
    mesh=_mesh,
    compiler_params=_sc_params,
    scratch_types=[
        pltpu.VMEM((NB, CE), jnp.int32),
        pltpu.VMEM((NB, CE), jnp.int32),
        pltpu.VMEM((128,), jnp.float32),
        pltpu.VMEM((2000,), jnp.float32),
        pltpu.VMEM_SHARED((N,), jnp.float32),
        pltpu.VMEM_SHARED((N,), jnp.float32),
    ],
)
def _deg_kernel(src_hbm, dst_hbm, dpart, src_v, dst_v, ones_v, z_v,
                acc_out, acc_in):
    cid = lax.axis_index("c")
    sid = lax.axis_index("s")
    wid = _worker_id()

    def fill_ones(i, _):
        ones_v[pl.ds(i * 16, 16)] = jnp.ones((16,), jnp.float32)
        return 0
    lax.fori_loop(0, 128 // 16, fill_ones, 0)

    def fill_z(i, _):
        z_v[pl.ds(i * 16, 16)] = jnp.zeros((16,), jnp.float32)
        return 0
    lax.fori_loop(0, 2000 // 16, fill_z, 0)

    @pl.when(sid < 5)
    def _():
        pltpu.sync_copy(z_v, acc_out.at[pl.ds(sid * 2000, 2000)])
        pltpu.sync_copy(z_v, acc_in.at[pl.ds(sid * 2000, 2000)])

    plsc.subcore_barrier()

    pltpu.sync_copy(src_hbm.at[wid], src_v)
    pltpu.sync_copy(dst_hbm.at[wid], dst_v)

    def body(j, _):
        pltpu.sync_copy(ones_v.at[pl.ds(0, CE)], acc_out.at[src_v.at[j]],
                        add=True)
        pltpu.sync_copy(ones_v.at[pl.ds(0, CE)], acc_in.at[dst_v.at[j]],
                        add=True)
        return 0
    lax.fori_loop(0, NB, body, 0)

    plsc.subcore_barrier()

    @pl.when(sid < 10)
    def _():
        pltpu.sync_copy(acc_out.at[pl.ds(sid * 1000, 1000)],
                        dpart.at[cid, 0, pl.ds(sid * 1000, 1000)])
        pltpu.sync_copy(acc_in.at[pl.ds(sid * 1000, 1000)],
                        dpart.at[cid, 1, pl.ds(sid * 1000, 1000)])


_BR = 1000


def _mlp_body(feat, w0, b0, w1, b1, w2, b2, dpo, dpi,
              s0_ref, h0a_ref, nin_ref, nout_ref):
    x = feat[...]
    h = jnp.maximum(jnp.dot(x, w0[...], preferred_element_type=jnp.float32)
                    + b0[...], 0.0)
    h = jnp.maximum(jnp.dot(h, w1[...], preferred_element_type=jnp.float32)
                    + b1[...], 0.0)
    h = jnp.dot(h, w2[...], preferred_element_type=jnp.float32) + b2[...]

    dout = jnp.maximum(dpo[0] + dpo[1], 1.0)
    din = jnp.maximum(dpi[0] + dpi[1], 1.0)
    no = lax.rsqrt(dout)
    ni = lax.rsqrt(din)

    s0_ref[...] = h * no
    h0a_ref[...] = ALPHA * h
    nin_ref[...] = (1.0 - ALPHA) * ni
    nout_ref[...] = no


def _mlp_kernel(features, W0, b0, W1, b1, W2, b2, dpo, dpi):
    grid = (N // _BR,)
    outm = jax.ShapeDtypeStruct((N, C), jnp.float32)
    outv = jax.ShapeDtypeStruct((N, 1), jnp.float32)
    return pl.pallas_call(
        _mlp_body,
        grid=grid,
        in_specs=[
            pl.BlockSpec((_BR, D), lambda i: (i, 0)),
            pl.BlockSpec((D, H), lambda i: (0, 0)),
            pl.BlockSpec((1, H), lambda i: (0, 0)),
            pl.BlockSpec((H, H), lambda i: (0, 0)),
            pl.BlockSpec((1, H), lambda i: (0, 0)),
            pl.BlockSpec((H, C), lambda i: (0, 0)),
            pl.BlockSpec((1, C), lambda i: (0, 0)),
            pl.BlockSpec((NC, _BR, 1), lambda i: (0, i, 0)),
            pl.BlockSpec((NC, _BR, 1), lambda i: (0, i, 0)),
        ],
        out_specs=[
            pl.BlockSpec((_BR, C), lambda i: (i, 0)),
            pl.BlockSpec((_BR, C), lambda i: (i, 0)),
            pl.BlockSpec((_BR, 1), lambda i: (i, 0)),
            pl.BlockSpec((_BR, 1), lambda i: (i, 0)),
        ],
        out_shape=[outm, outm, outv, outv],
    )(features, W0, b0.reshape(1, H), W1, b1.reshape(1, H),
      W2, b2.reshape(1, C), dpo, dpi)


@functools.partial(
    pl.kernel,
    out_type=jax.ShapeDtypeStruct((NC * N, HC), jnp.float32),
    mesh=_mesh,
    compiler_params=_sc_params,
    scratch_types=[
        pltpu.VMEM((NGB, GB), jnp.int32),
        pltpu.VMEM((NGB, GB), jnp.int32),
        pltpu.VMEM((NBUF, GB, HC), jnp.float32),
        pltpu.VMEM((RCH, HC), jnp.float32),
        pltpu.VMEM_SHARED((N, HC), jnp.float32),
        pltpu.SemaphoreType.DMA((NBUF,)),
        pltpu.SemaphoreType.DMA((NBUF,)),
    ],
)
def _prop_kernel(s_init, h0a2, ninm, noutm, srcr, dstr, s_buf,
                 src_v, dst_v, rows_v, zch, acc, gsem, ssem):
    accv = rows_v.at[4]
    h0av = rows_v.at[5]
    ninv = rows_v.at[6]
    noutv = rows_v.at[7]
    outv = rows_v.at[4]
    cid = lax.axis_index("c")
    sid = lax.axis_index("s")
    base_row = sid * RPT

    off = cid * N

    pltpu.sync_copy(srcr.at[cid, sid], src_v)
    pltpu.sync_copy(dstr.at[sid], dst_v)

    def fz(i, _):
        zch[i // (HC // 16), pl.ds((i % (HC // 16)) * 16, 16)] = (
            jnp.zeros((16,), jnp.float32))
        return 0
    lax.fori_loop(0, RCH * (HC // 16), fz, 0)

    def cinit(t, _):
        r0 = off + base_row + t * RCH
        pltpu.sync_copy(s_init.at[pl.ds(r0, RCH)], accv)
        pltpu.sync_copy(accv, s_buf.at[pl.ds(r0, RCH)])
        pltpu.sync_copy(zch, acc.at[pl.ds(base_row + t * RCH, RCH)])
        return 0
    lax.fori_loop(0, NCH, cinit, 0)

    plsc.subcore_barrier()

    def step(k, _):
        for p in range(LA):
            pltpu.async_copy(s_buf.at[src_v.at[p]], rows_v.at[p],
                             gsem.at[p])

        def body(j, _):
            b = j % NBUF
            ahead = (j + LA) % NBUF

            @pl.when(j >= NBUF - LA)
            def _():
                pltpu.make_async_copy(rows_v.at[ahead],
                                      acc.at[dst_v.at[j - (NBUF - LA)]],
                                      ssem.at[ahead]).wait()

            @pl.when(j + LA < NGB)
            def _():
                pltpu.async_copy(s_buf.at[src_v.at[j + LA]],
                                 rows_v.at[ahead], gsem.at[ahead])

            pltpu.make_async_copy(s_buf.at[src_v.at[j]], rows_v.at[b],
                                  gsem.at[b]).wait()
            pltpu.async_copy(rows_v.at[b], acc.at[dst_v.at[j]], ssem.at[b],
                             add=True)
            return 0
        lax.fori_loop(0, NGB, body, 0)

        def drain(d, _):
            j = NGB - (NBUF - LA) + d
            pltpu.make_async_copy(rows_v.at[j % NBUF],
                                  acc.at[dst_v.at[j]],
                                  ssem.at[j % NBUF]).wait()
            return 0
        lax.fori_loop(0, NBUF - LA, drain, 0)

        plsc.subcore_barrier()

        last = k == K_PROP - 1

        def comb(t, _):
            r0 = base_row + t * RCH
            pltpu.sync_copy(acc.at[pl.ds(r0, RCH)], accv)
            pltpu.sync_copy(zch, acc.at[pl.ds(r0, RCH)])
            pltpu.sync_copy(h0a2.at[pl.ds(off + r0, RCH)], h0av)
            pltpu.sync_copy(ninm.at[pl.ds(off + r0, RCH)], ninv)
            pltpu.sync_copy(noutm.at[pl.ds(off + r0, RCH)], noutv)

            def rowloop(i, _):
                r = i // (HC // 16)
                sl = pl.ds((i % (HC // 16)) * 16, 16)
                no = jnp.where(last, 1.0, noutv[r, sl])
                outv[r, sl] = (accv[r, sl] * ninv[r, sl]
                               + h0av[r, sl]) * no
                return 0
            lax.fori_loop(0, RCH * (HC // 16), rowloop, 0)

            pltpu.sync_copy(outv, s_buf.at[pl.ds(off + r0, RCH)])
            return 0
        lax.fori_loop(0, NCH, comb, 0)

        plsc.subcore_barrier()
        return 0
    lax.fori_loop(0, K_PROP, step, 0)


def _split_cols(x):
    return x.reshape(N, NC, HC).transpose(1, 0, 2).reshape(NC * N, HC)


def kernel(features, edge_index, W0, b0, W1, b1, W2, b2):
    src = edge_index[0]
    dst = edge_index[1]

    dpart = _deg_kernel(src.reshape(NW, NB, CE), dst.reshape(NW, NB, CE))
    dpo = dpart[:, 0, :].reshape(NC, N, 1)
    dpi = dpart[:, 1, :].reshape(NC, N, 1)

    s0, h0a, nin1, nout = _mlp_kernel(features, W0, b0, W1, b1, W2, b2,
                                      dpo, dpi)

    ninm = _split_cols(jnp.broadcast_to(nin1, (N, C)))
    noutm = _split_cols(jnp.broadcast_to(nout, (N, C)))
    src2 = jnp.stack([src, src + N]).reshape(NC, NS, NGB, GB)
    s_buf = _prop_kernel(_split_cols(s0), _split_cols(h0a), ninm, noutm,
                         src2, dst.reshape(NS, NGB, GB))

    return s_buf.reshape(NC, N, HC).transpose(1, 0, 2).reshape(N, C)


# speedup vs baseline: 1.2831x; 1.1937x over previous
import functools

import jax
import jax.numpy as jnp
from jax import lax
from jax.experimental import pallas as pl
from jax.experimental.pallas import tpu as pltpu
from jax.experimental.pallas import tpu_sc as plsc

N = 10000
E = 320000
D = 128
H = 256
C = 64
K_PROP = 10
ALPHA = 0.1

NC = 2
NS = 16
NW = NC * NS
EPW = E // NW
CE = 125
NB = EPW // CE

HC = C // NC
TPE = E // NS
GB = 125
NGB = TPE // GB
NBUF = 8
RPT = N // NS
RCH = 125
NCH = RPT // RCH

_mesh = plsc.VectorSubcoreMesh(core_axis_name="c", subcore_axis_name="s",
                               num_cores=NC, num_subcores=NS)
_sc_params = pltpu.CompilerParams(use_tc_tiling_on_sc=False)


def _worker_id():
    return lax.axis_index("s") * NC + lax.axis_index("c")


@functools.partial(
    pl.kernel,
    out_type=jax.ShapeDtypeStruct((NC, 2, N), jnp.float32),
    mesh=_mesh,
    compiler_params=_sc_params,
    scratch_types=[
        pltpu.VMEM((NB, CE), jnp.int32),
        pltpu.VMEM((NB, CE), jnp.int32),
        pltpu.VMEM((128,), jnp.float32),
        pltpu.VMEM((2000,), jnp.float32),
        pltpu.VMEM_SHARED((N,), jnp.float32),
        pltpu.VMEM_SHARED((N,), jnp.float32),
    ],
)
def _deg_kernel(src_hbm, dst_hbm, dpart, src_v, dst_v, ones_v, z_v,
                acc_out, acc_in):
    cid = lax.axis_index("c")
    sid = lax.axis_index("s")
    wid = _worker_id()

    def fill_ones(i, _):
        ones_v[pl.ds(i * 16, 16)] = jnp.ones((16,), jnp.float32)
        return 0
    lax.fori_loop(0, 128 // 16, fill_ones, 0)

    def fill_z(i, _):
        z_v[pl.ds(i * 16, 16)] = jnp.zeros((16,), jnp.float32)
        return 0
    lax.fori_loop(0, 2000 // 16, fill_z, 0)

    @pl.when(sid < 5)
    def _():
        pltpu.sync_copy(z_v, acc_out.at[pl.ds(sid * 2000, 2000)])
        pltpu.sync_copy(z_v, acc_in.at[pl.ds(sid * 2000, 2000)])

    plsc.subcore_barrier()

    pltpu.sync_copy(src_hbm.at[wid], src_v)
    pltpu.sync_copy(dst_hbm.at[wid], dst_v)

    def body(j, _):
        pltpu.sync_copy(ones_v.at[pl.ds(0, CE)], acc_out.at[src_v.at[j]],
                        add=True)
        pltpu.sync_copy(ones_v.at[pl.ds(0, CE)], acc_in.at[dst_v.at[j]],
                        add=True)
        return 0
    lax.fori_loop(0, NB, body, 0)

    plsc.subcore_barrier()

    @pl.when(sid < 10)
    def _():
        pltpu.sync_copy(acc_out.at[pl.ds(sid * 1000, 1000)],
                        dpart.at[cid, 0, pl.ds(sid * 1000, 1000)])
        pltpu.sync_copy(acc_in.at[pl.ds(sid * 1000, 1000)],
                        dpart.at[cid, 1, pl.ds(sid * 1000, 1000)])


_BR = 1000


def _mlp_body(feat, w0, b0, w1, b1, w2, b2, dpo, dpi,
              s0_ref, h0a_ref, nin_ref, nout_ref):
    x = feat[...]
    h = jnp.maximum(jnp.dot(x, w0[...], preferred_element_type=jnp.float32)
                    + b0[...], 0.0)
    h = jnp.maximum(jnp.dot(h, w1[...], preferred_element_type=jnp.float32)
                    + b1[...], 0.0)
    h = jnp.dot(h, w2[...], preferred_element_type=jnp.float32) + b2[...]

    dout = jnp.maximum(dpo[0] + dpo[1], 1.0)
    din = jnp.maximum(dpi[0] + dpi[1], 1.0)
    no = lax.rsqrt(dout)
    ni = lax.rsqrt(din)

    s0_ref[...] = h * no
    h0a_ref[...] = ALPHA * h
    nin_ref[...] = (1.0 - ALPHA) * ni
    nout_ref[...] = no


def _mlp_kernel(features, W0, b0, W1, b1, W2, b2, dpo, dpi):
    grid = (N // _BR,)
    outm = jax.ShapeDtypeStruct((N, C), jnp.float32)
    outv = jax.ShapeDtypeStruct((N, 1), jnp.float32)
    return pl.pallas_call(
        _mlp_body,
        grid=grid,
        in_specs=[
            pl.BlockSpec((_BR, D), lambda i: (i, 0)),
            pl.BlockSpec((D, H), lambda i: (0, 0)),
            pl.BlockSpec((1, H), lambda i: (0, 0)),
            pl.BlockSpec((H, H), lambda i: (0, 0)),
            pl.BlockSpec((1, H), lambda i: (0, 0)),
            pl.BlockSpec((H, C), lambda i: (0, 0)),
            pl.BlockSpec((1, C), lambda i: (0, 0)),
            pl.BlockSpec((NC, _BR, 1), lambda i: (0, i, 0)),
            pl.BlockSpec((NC, _BR, 1), lambda i: (0, i, 0)),
        ],
        out_specs=[
            pl.BlockSpec((_BR, C), lambda i: (i, 0)),
            pl.BlockSpec((_BR, C), lambda i: (i, 0)),
            pl.BlockSpec((_BR, 1), lambda i: (i, 0)),
            pl.BlockSpec((_BR, 1), lambda i: (i, 0)),
        ],
        out_shape=[outm, outm, outv, outv],
    )(features, W0, b0.reshape(1, H), W1, b1.reshape(1, H),
      W2, b2.reshape(1, C), dpo, dpi)


@functools.partial(
    pl.kernel,
    out_type=jax.ShapeDtypeStruct((NC * N, HC), jnp.float32),
    mesh=_mesh,
    compiler_params=_sc_params,
    scratch_types=[
        pltpu.VMEM((NGB, GB), jnp.int32),
        pltpu.VMEM((NGB, GB), jnp.int32),
        pltpu.VMEM((NBUF, GB, HC), jnp.float32),
        pltpu.VMEM((RCH, HC), jnp.float32),
        pltpu.VMEM_SHARED((N, HC), jnp.float32),
        pltpu.SemaphoreType.DMA((NBUF,)),
        pltpu.SemaphoreType.DMA((NBUF,)),
    ],
)
def _prop_kernel(s_init, h0a2, ninm, noutm, srcr, dstr, s_buf,
                 src_v, dst_v, rows_v, zch, acc, gsem, ssem):
    accv = rows_v.at[4]
    cid = lax.axis_index("c")
    sid = lax.axis_index("s")
    base_row = sid * RPT

    off = cid * N

    pltpu.sync_copy(srcr.at[cid, sid], src_v)
    pltpu.sync_copy(dstr.at[sid], dst_v)

    def fz(i, _):
        zch[i // (HC // 16), pl.ds((i % (HC // 16)) * 16, 16)] = (
            jnp.zeros((16,), jnp.float32))
        return 0
    lax.fori_loop(0, RCH * (HC // 16), fz, 0)

    def cinit(t, _):
        r0 = off + base_row + t * RCH
        pltpu.sync_copy(s_init.at[pl.ds(r0, RCH)], accv)
        pltpu.sync_copy(accv, s_buf.at[pl.ds(r0, RCH)])
        pltpu.sync_copy(zch, acc.at[pl.ds(base_row + t * RCH, RCH)])
        return 0
    lax.fori_loop(0, NCH, cinit, 0)

    plsc.subcore_barrier()

    def step(k, _):
        for p in range(NBUF - 1):
            pltpu.async_copy(s_buf.at[src_v.at[p]], rows_v.at[p],
                             gsem.at[p])

        def body(j, _):
            b = j % NBUF
            ahead = (j + NBUF - 1) % NBUF

            @pl.when(j >= 1)
            def _():
                pltpu.make_async_copy(rows_v.at[ahead],
                                      acc.at[dst_v.at[j - 1]],
                                      ssem.at[ahead]).wait()

            @pl.when(j + NBUF - 1 < NGB)
            def _():
                pltpu.async_copy(s_buf.at[src_v.at[j + NBUF - 1]],
                                 rows_v.at[ahead], gsem.at[ahead])

            pltpu.make_async_copy(s_buf.at[src_v.at[j]], rows_v.at[b],
                                  gsem.at[b]).wait()
            pltpu.async_copy(rows_v.at[b], acc.at[dst_v.at[j]], ssem.at[b],
                             add=True)
            return 0
        lax.fori_loop(0, NGB, body, 0)

        pltpu.make_async_copy(rows_v.at[(NGB - 1) % NBUF],
                              acc.at[dst_v.at[NGB - 1]],
                              ssem.at[(NGB - 1) % NBUF]).wait()

        plsc.subcore_barrier()

        last = k == K_PROP - 1

        def issue_loads(t, g4):
            r0 = base_row + t * RCH
            pltpu.async_copy(acc.at[pl.ds(r0, RCH)], rows_v.at[g4 + 0],
                             gsem.at[g4 + 0])
            pltpu.async_copy(h0a2.at[pl.ds(off + r0, RCH)],
                             rows_v.at[g4 + 1], gsem.at[g4 + 1])
            pltpu.async_copy(ninm.at[pl.ds(off + r0, RCH)],
                             rows_v.at[g4 + 2], gsem.at[g4 + 2])
            pltpu.async_copy(noutm.at[pl.ds(off + r0, RCH)],
                             rows_v.at[g4 + 3], gsem.at[g4 + 3])

        issue_loads(0, 0)

        def comb(t, _):
            g4 = (t % 2) * 4
            og4 = ((t + 1) % 2) * 4
            r0 = base_row + t * RCH

            @pl.when(t >= 1)
            def _():
                pltpu.make_async_copy(
                    rows_v.at[og4],
                    s_buf.at[pl.ds(off + r0 - RCH, RCH)],
                    ssem.at[og4]).wait()

            @pl.when(t + 1 < NCH)
            def _():
                issue_loads(t + 1, og4)

            for q in range(4):
                pltpu.make_async_copy(acc.at[pl.ds(r0, RCH)],
                                      rows_v.at[g4 + q],
                                      gsem.at[g4 + q]).wait()

            pltpu.sync_copy(zch, acc.at[pl.ds(r0, RCH)])

            def rowloop(i, _):
                r = i // (HC // 16)
                sl = pl.ds((i % (HC // 16)) * 16, 16)
                no = jnp.where(last, 1.0, rows_v[g4 + 3, r, sl])
                rows_v[g4, r, sl] = (rows_v[g4, r, sl]
                                     * rows_v[g4 + 2, r, sl]
                                     + rows_v[g4 + 1, r, sl]) * no
                return 0
            lax.fori_loop(0, RCH * (HC // 16), rowloop, 0)

            pltpu.async_copy(rows_v.at[g4], s_buf.at[pl.ds(off + r0, RCH)],
                             ssem.at[g4])
            return 0
        lax.fori_loop(0, NCH, comb, 0)

        g4l = ((NCH - 1) % 2) * 4
        pltpu.make_async_copy(
            rows_v.at[g4l],
            s_buf.at[pl.ds(off + base_row + (NCH - 1) * RCH, RCH)],
            ssem.at[g4l]).wait()

        plsc.subcore_barrier()
        return 0
    lax.fori_loop(0, K_PROP, step, 0)


def _split_cols(x):
    return x.reshape(N, NC, HC).transpose(1, 0, 2).reshape(NC * N, HC)


def kernel(features, edge_index, W0, b0, W1, b1, W2, b2):
    src = edge_index[0]
    dst = edge_index[1]

    dpart = _deg_kernel(src.reshape(NW, NB, CE), dst.reshape(NW, NB, CE))
    dpo = dpart[:, 0, :].reshape(NC, N, 1)
    dpi = dpart[:, 1, :].reshape(NC, N, 1)

    s0, h0a, nin1, nout = _mlp_kernel(features, W0, b0, W1, b1, W2, b2,
                                      dpo, dpi)

    ninm = _split_cols(jnp.broadcast_to(nin1, (N, C)))
    noutm = _split_cols(jnp.broadcast_to(nout, (N, C)))
    src2 = jnp.stack([src, src + N]).reshape(NC, NS, NGB, GB)
    s_buf = _prop_kernel(_split_cols(s0), _split_cols(h0a), ninm, noutm,
                         src2, dst.reshape(NS, NGB, GB))

    return s_buf.reshape(NC, N, HC).transpose(1, 0, 2).reshape(N, C)


# speedup vs baseline: 1.3712x; 1.0687x over previous
import functools

import jax
import jax.numpy as jnp
from jax import lax
from jax.experimental import pallas as pl
from jax.experimental.pallas import tpu as pltpu
from jax.experimental.pallas import tpu_sc as plsc

N = 10000
E = 320000
D = 128
H = 256
C = 64
K_PROP = 10
ALPHA = 0.1

NC = 2
NS = 16
NW = NC * NS
EPW = E // NW
CE = 125
NB = EPW // CE

HC = C // NC
TPE = E // NS
GB = 125
NGB = TPE // GB
NBUF = 8
RPT = N // NS
RCH = 125
NCH = RPT // RCH

_mesh = plsc.VectorSubcoreMesh(core_axis_name="c", subcore_axis_name="s",
                               num_cores=NC, num_subcores=NS)
_sc_params = pltpu.CompilerParams(use_tc_tiling_on_sc=False)


def _worker_id():
    return lax.axis_index("s") * NC + lax.axis_index("c")


@functools.partial(
    pl.kernel,
    out_type=jax.ShapeDtypeStruct((NC, 2, N), jnp.float32),
    mesh=_mesh,
    compiler_params=_sc_params,
    scratch_types=[
        pltpu.VMEM((NB, CE), jnp.int32),
        pltpu.VMEM((NB, CE), jnp.int32),
        pltpu.VMEM((128,), jnp.float32),
        pltpu.VMEM((2000,), jnp.float32),
        pltpu.VMEM_SHARED((N,), jnp.float32),
        pltpu.VMEM_SHARED((N,), jnp.float32),
    ],
)
def _deg_kernel(src_hbm, dst_hbm, dpart, src_v, dst_v, ones_v, z_v,
                acc_out, acc_in):
    cid = lax.axis_index("c")
    sid = lax.axis_index("s")
    wid = _worker_id()

    def fill_ones(i, _):
        ones_v[pl.ds(i * 16, 16)] = jnp.ones((16,), jnp.float32)
        return 0
    lax.fori_loop(0, 128 // 16, fill_ones, 0)

    def fill_z(i, _):
        z_v[pl.ds(i * 16, 16)] = jnp.zeros((16,), jnp.float32)
        return 0
    lax.fori_loop(0, 2000 // 16, fill_z, 0)

    @pl.when(sid < 5)
    def _():
        pltpu.sync_copy(z_v, acc_out.at[pl.ds(sid * 2000, 2000)])
        pltpu.sync_copy(z_v, acc_in.at[pl.ds(sid * 2000, 2000)])

    plsc.subcore_barrier()

    pltpu.sync_copy(src_hbm.at[wid], src_v)
    pltpu.sync_copy(dst_hbm.at[wid], dst_v)

    def body(j, _):
        pltpu.sync_copy(ones_v.at[pl.ds(0, CE)], acc_out.at[src_v.at[j]],
                        add=True)
        pltpu.sync_copy(ones_v.at[pl.ds(0, CE)], acc_in.at[dst_v.at[j]],
                        add=True)
        return 0
    lax.fori_loop(0, NB, body, 0)

    plsc.subcore_barrier()

    @pl.when(sid < 10)
    def _():
        pltpu.sync_copy(acc_out.at[pl.ds(sid * 1000, 1000)],
                        dpart.at[cid, 0, pl.ds(sid * 1000, 1000)])
        pltpu.sync_copy(acc_in.at[pl.ds(sid * 1000, 1000)],
                        dpart.at[cid, 1, pl.ds(sid * 1000, 1000)])


_BR = 1000


def _mlp_body(feat, w0, b0, w1, b1, w2, b2, h0a_ref, h0b_ref):
    x = feat[...]
    h = jnp.maximum(jnp.dot(x, w0[...], preferred_element_type=jnp.float32)
                    + b0[...], 0.0)
    h = jnp.maximum(jnp.dot(h, w1[...], preferred_element_type=jnp.float32)
                    + b1[...], 0.0)
    h = jnp.dot(h, w2[...], preferred_element_type=jnp.float32) + b2[...]
    h0a_ref[...] = h[:, :HC]
    h0b_ref[...] = h[:, HC:]


def _mlp_kernel(features, W0, b0, W1, b1, W2, b2):
    grid = (N // _BR,)
    outh = jax.ShapeDtypeStruct((N, HC), jnp.float32)
    return pl.pallas_call(
        _mlp_body,
        grid=grid,
        in_specs=[
            pl.BlockSpec((_BR, D), lambda i: (i, 0)),
            pl.BlockSpec((D, H), lambda i: (0, 0)),
            pl.BlockSpec((1, H), lambda i: (0, 0)),
            pl.BlockSpec((H, H), lambda i: (0, 0)),
            pl.BlockSpec((1, H), lambda i: (0, 0)),
            pl.BlockSpec((H, C), lambda i: (0, 0)),
            pl.BlockSpec((1, C), lambda i: (0, 0)),
        ],
        out_specs=[
            pl.BlockSpec((_BR, HC), lambda i: (i, 0)),
            pl.BlockSpec((_BR, HC), lambda i: (i, 0)),
        ],
        out_shape=[outh, outh],
    )(features, W0, b0.reshape(1, H), W1, b1.reshape(1, H),
      W2, b2.reshape(1, C))


def _norm_body(dpo, dpi, nin_ref, nout_ref):
    dout = jnp.maximum(dpo[0] + dpo[1], 1.0)
    din = jnp.maximum(dpi[0] + dpi[1], 1.0)
    no = lax.rsqrt(dout)
    ni = lax.rsqrt(din)
    nin_ref[...] = jnp.broadcast_to((1.0 - ALPHA) * ni, (_BR, HC))
    nout_ref[...] = jnp.broadcast_to(no, (_BR, HC))


def _norm_kernel(dpo, dpi):
    grid = (N // _BR,)
    outm = jax.ShapeDtypeStruct((N, HC), jnp.float32)
    return pl.pallas_call(
        _norm_body,
        grid=grid,
        in_specs=[
            pl.BlockSpec((NC, _BR, 1), lambda i: (0, i, 0)),
            pl.BlockSpec((NC, _BR, 1), lambda i: (0, i, 0)),
        ],
        out_specs=[
            pl.BlockSpec((_BR, HC), lambda i: (i, 0)),
            pl.BlockSpec((_BR, HC), lambda i: (i, 0)),
        ],
        out_shape=[outm, outm],
    )(dpo, dpi)


@functools.partial(
    pl.kernel,
    out_type=jax.ShapeDtypeStruct((NC * N, HC), jnp.float32),
    mesh=_mesh,
    compiler_params=_sc_params,
    scratch_types=[
        pltpu.VMEM((NGB, GB), jnp.int32),
        pltpu.VMEM((NGB, GB), jnp.int32),
        pltpu.VMEM((NBUF, GB, HC), jnp.float32),
        pltpu.VMEM((RCH, HC), jnp.float32),
        pltpu.VMEM_SHARED((N, HC), jnp.float32),
        pltpu.SemaphoreType.DMA((NBUF,)),
        pltpu.SemaphoreType.DMA((NBUF,)),
    ],
)
def _prop_kernel(h0_a, h0_b, ninm, noutm, srcr, dstr, s_buf,
                 src_v, dst_v, rows_v, zch, acc, gsem, ssem):
    accv = rows_v.at[4]
    cid = lax.axis_index("c")
    sid = lax.axis_index("s")
    base_row = sid * RPT

    off = cid * N

    pltpu.sync_copy(srcr.at[cid, sid], src_v)
    pltpu.sync_copy(dstr.at[sid], dst_v)

    def fz(i, _):
        zch[i // (HC // 16), pl.ds((i % (HC // 16)) * 16, 16)] = (
            jnp.zeros((16,), jnp.float32))
        return 0
    lax.fori_loop(0, RCH * (HC // 16), fz, 0)

    def cinit(t, _):
        r0 = base_row + t * RCH

        @pl.when(cid == 0)
        def _():
            pltpu.sync_copy(h0_a.at[pl.ds(r0, RCH)], rows_v.at[1])

        @pl.when(cid == 1)
        def _():
            pltpu.sync_copy(h0_b.at[pl.ds(r0, RCH)], rows_v.at[1])

        pltpu.sync_copy(noutm.at[pl.ds(r0, RCH)], rows_v.at[3])

        def iloop(i, _):
            r = i // (HC // 16)
            sl = pl.ds((i % (HC // 16)) * 16, 16)
            rows_v[1, r, sl] = rows_v[1, r, sl] * rows_v[3, r, sl]
            return 0
        lax.fori_loop(0, RCH * (HC // 16), iloop, 0)

        pltpu.sync_copy(rows_v.at[1], s_buf.at[pl.ds(off + r0, RCH)])
        pltpu.sync_copy(zch, acc.at[pl.ds(r0, RCH)])
        return 0
    lax.fori_loop(0, NCH, cinit, 0)

    plsc.subcore_barrier()

    def step(k, _):
        for p in range(NBUF - 1):
            pltpu.async_copy(s_buf.at[src_v.at[p]], rows_v.at[p],
                             gsem.at[p])

        def body(j, _):
            b = j % NBUF
            ahead = (j + NBUF - 1) % NBUF

            @pl.when(j >= 1)
            def _():
                pltpu.make_async_copy(rows_v.at[ahead],
                                      acc.at[dst_v.at[j - 1]],
                                      ssem.at[ahead]).wait()

            @pl.when(j + NBUF - 1 < NGB)
            def _():
                pltpu.async_copy(s_buf.at[src_v.at[j + NBUF - 1]],
                                 rows_v.at[ahead], gsem.at[ahead])

            pltpu.make_async_copy(s_buf.at[src_v.at[j]], rows_v.at[b],
                                  gsem.at[b]).wait()
            pltpu.async_copy(rows_v.at[b], acc.at[dst_v.at[j]], ssem.at[b],
                             add=True)
            return 0
        lax.fori_loop(0, NGB, body, 0)

        pltpu.make_async_copy(rows_v.at[(NGB - 1) % NBUF],
                              acc.at[dst_v.at[NGB - 1]],
                              ssem.at[(NGB - 1) % NBUF]).wait()

        plsc.subcore_barrier()

        last = k == K_PROP - 1

        def issue_loads(t, g4):
            r0 = base_row + t * RCH

            @pl.when(cid == 0)
            def _():
                pltpu.async_copy(h0_a.at[pl.ds(r0, RCH)],
                                 rows_v.at[g4 + 1], gsem.at[g4 + 1])

            @pl.when(cid == 1)
            def _():
                pltpu.async_copy(h0_b.at[pl.ds(r0, RCH)],
                                 rows_v.at[g4 + 1], gsem.at[g4 + 1])

            pltpu.async_copy(acc.at[pl.ds(r0, RCH)], rows_v.at[g4 + 0],
                             gsem.at[g4 + 0])
            pltpu.async_copy(ninm.at[pl.ds(r0, RCH)],
                             rows_v.at[g4 + 2], gsem.at[g4 + 2])
            pltpu.async_copy(noutm.at[pl.ds(r0, RCH)],
                             rows_v.at[g4 + 3], gsem.at[g4 + 3])

        issue_loads(0, 0)

        def comb(t, _):
            g4 = (t % 2) * 4
            og4 = ((t + 1) % 2) * 4
            r0 = base_row + t * RCH

            @pl.when(t >= 1)
            def _():
                pltpu.make_async_copy(
                    rows_v.at[og4],
                    s_buf.at[pl.ds(off + r0 - RCH, RCH)],
                    ssem.at[og4]).wait()

            @pl.when(t + 1 < NCH)
            def _():
                issue_loads(t + 1, og4)

            for q in range(4):
                pltpu.make_async_copy(acc.at[pl.ds(r0, RCH)],
                                      rows_v.at[g4 + q],
                                      gsem.at[g4 + q]).wait()

            pltpu.sync_copy(zch, acc.at[pl.ds(r0, RCH)])

            def rowloop(i, _):
                r = i // (HC // 16)
                sl = pl.ds((i % (HC // 16)) * 16, 16)
                no = jnp.where(last, 1.0, rows_v[g4 + 3, r, sl])
                rows_v[g4, r, sl] = (rows_v[g4, r, sl]
                                     * rows_v[g4 + 2, r, sl]
                                     + ALPHA * rows_v[g4 + 1, r, sl]) * no
                return 0
            lax.fori_loop(0, RCH * (HC // 16), rowloop, 0)

            pltpu.async_copy(rows_v.at[g4], s_buf.at[pl.ds(off + r0, RCH)],
                             ssem.at[g4])
            return 0
        lax.fori_loop(0, NCH, comb, 0)

        g4l = ((NCH - 1) % 2) * 4
        pltpu.make_async_copy(
            rows_v.at[g4l],
            s_buf.at[pl.ds(off + base_row + (NCH - 1) * RCH, RCH)],
            ssem.at[g4l]).wait()

        plsc.subcore_barrier()
        return 0
    lax.fori_loop(0, K_PROP, step, 0)


def kernel(features, edge_index, W0, b0, W1, b1, W2, b2):
    src = edge_index[0]
    dst = edge_index[1]

    dpart = _deg_kernel(src.reshape(NW, NB, CE), dst.reshape(NW, NB, CE))
    dpo = dpart[:, 0, :].reshape(NC, N, 1)
    dpi = dpart[:, 1, :].reshape(NC, N, 1)

    h0_a, h0_b = _mlp_kernel(features, W0, b0, W1, b1, W2, b2)
    ninm, noutm = _norm_kernel(dpo, dpi)

    src2 = jnp.stack([src, src + N]).reshape(NC, NS, NGB, GB)
    s_buf = _prop_kernel(h0_a, h0_b, ninm, noutm,
                         src2, dst.reshape(NS, NGB, GB))

    return s_buf.reshape(NC, N, HC).transpose(1, 0, 2).reshape(N, C)


# speedup vs baseline: 1.3927x; 1.0157x over previous
import functools

import jax
import jax.numpy as jnp
from jax import lax
from jax.experimental import pallas as pl
from jax.experimental.pallas import tpu as pltpu
from jax.experimental.pallas import tpu_sc as plsc

N = 10000
E = 320000
D = 128
H = 256
C = 64
K_PROP = 10
ALPHA = 0.1

NC = 2
NS = 16
NW = NC * NS
EPW = E // NW
CE = 125
NB = EPW // CE

HC = C // NC
TPE = E // NS
GB = 125
NGB = TPE // GB
NBUF = 8
RPT = N // NS
RCH = 125
NCH = RPT // RCH

_mesh = plsc.VectorSubcoreMesh(core_axis_name="c", subcore_axis_name="s",
                               num_cores=NC, num_subcores=NS)
_sc_params = pltpu.CompilerParams(use_tc_tiling_on_sc=False)


def _worker_id():
    return lax.axis_index("s") * NC + lax.axis_index("c")


@functools.partial(
    pl.kernel,
    out_type=jax.ShapeDtypeStruct((NC, 2, N), jnp.float32),
    mesh=_mesh,
    compiler_params=_sc_params,
    scratch_types=[
        pltpu.VMEM((NB, CE), jnp.int32),
        pltpu.VMEM((NB, CE), jnp.int32),
        pltpu.VMEM((128,), jnp.float32),
        pltpu.VMEM((2000,), jnp.float32),
        pltpu.VMEM_SHARED((N,), jnp.float32),
        pltpu.VMEM_SHARED((N,), jnp.float32),
        pltpu.SemaphoreType.DMA((4,)),
    ],
)
def _deg_kernel(src_hbm, dst_hbm, dpart, src_v, dst_v, ones_v, z_v,
                acc_out, acc_in, dsem):
    cid = lax.axis_index("c")
    sid = lax.axis_index("s")
    wid = _worker_id()

    def fill_ones(i, _):
        ones_v[pl.ds(i * 16, 16)] = jnp.ones((16,), jnp.float32)
        return 0
    lax.fori_loop(0, 128 // 16, fill_ones, 0)

    def fill_z(i, _):
        z_v[pl.ds(i * 16, 16)] = jnp.zeros((16,), jnp.float32)
        return 0
    lax.fori_loop(0, 2000 // 16, fill_z, 0)

    @pl.when(sid < 5)
    def _():
        pltpu.sync_copy(z_v, acc_out.at[pl.ds(sid * 2000, 2000)])
        pltpu.sync_copy(z_v, acc_in.at[pl.ds(sid * 2000, 2000)])

    plsc.subcore_barrier()

    pltpu.sync_copy(src_hbm.at[wid], src_v)
    pltpu.sync_copy(dst_hbm.at[wid], dst_v)

    def body(j, _):
        p2 = (j % 2) * 2

        @pl.when(j >= 2)
        def _():
            pltpu.make_async_copy(ones_v.at[pl.ds(0, CE)],
                                  acc_out.at[src_v.at[j - 2]],
                                  dsem.at[p2]).wait()
            pltpu.make_async_copy(ones_v.at[pl.ds(0, CE)],
                                  acc_in.at[dst_v.at[j - 2]],
                                  dsem.at[p2 + 1]).wait()

        pltpu.async_copy(ones_v.at[pl.ds(0, CE)], acc_out.at[src_v.at[j]],
                         dsem.at[p2], add=True)
        pltpu.async_copy(ones_v.at[pl.ds(0, CE)], acc_in.at[dst_v.at[j]],
                         dsem.at[p2 + 1], add=True)
        return 0
    lax.fori_loop(0, NB, body, 0)

    def dega_drain(d, _):
        j = NB - 2 + d
        p2 = (j % 2) * 2
        pltpu.make_async_copy(ones_v.at[pl.ds(0, CE)],
                              acc_out.at[src_v.at[j]], dsem.at[p2]).wait()
        pltpu.make_async_copy(ones_v.at[pl.ds(0, CE)],
                              acc_in.at[dst_v.at[j]],
                              dsem.at[p2 + 1]).wait()
        return 0
    lax.fori_loop(0, 2, dega_drain, 0)

    plsc.subcore_barrier()

    @pl.when(sid < 10)
    def _():
        pltpu.sync_copy(acc_out.at[pl.ds(sid * 1000, 1000)],
                        dpart.at[cid, 0, pl.ds(sid * 1000, 1000)])
        pltpu.sync_copy(acc_in.at[pl.ds(sid * 1000, 1000)],
                        dpart.at[cid, 1, pl.ds(sid * 1000, 1000)])


_BR = 1000


def _mlp_body(feat, w0, b0, w1, b1, w2, b2, h0a_ref, h0b_ref):
    x = feat[...]
    h = jnp.maximum(jnp.dot(x, w0[...], preferred_element_type=jnp.float32)
                    + b0[...], 0.0)
    h = jnp.maximum(jnp.dot(h, w1[...], preferred_element_type=jnp.float32)
                    + b1[...], 0.0)
    h = jnp.dot(h, w2[...], preferred_element_type=jnp.float32) + b2[...]
    h0a_ref[...] = h[:, :HC]
    h0b_ref[...] = h[:, HC:]


def _mlp_kernel(features, W0, b0, W1, b1, W2, b2):
    grid = (N // _BR,)
    outh = jax.ShapeDtypeStruct((N, HC), jnp.float32)
    return pl.pallas_call(
        _mlp_body,
        grid=grid,
        in_specs=[
            pl.BlockSpec((_BR, D), lambda i: (i, 0)),
            pl.BlockSpec((D, H), lambda i: (0, 0)),
            pl.BlockSpec((1, H), lambda i: (0, 0)),
            pl.BlockSpec((H, H), lambda i: (0, 0)),
            pl.BlockSpec((1, H), lambda i: (0, 0)),
            pl.BlockSpec((H, C), lambda i: (0, 0)),
            pl.BlockSpec((1, C), lambda i: (0, 0)),
        ],
        out_specs=[
            pl.BlockSpec((_BR, HC), lambda i: (i, 0)),
            pl.BlockSpec((_BR, HC), lambda i: (i, 0)),
        ],
        out_shape=[outh, outh],
    )(features, W0, b0.reshape(1, H), W1, b1.reshape(1, H),
      W2, b2.reshape(1, C))


def _norm_body(dpo, dpi, nin_ref, nout_ref):
    dout = jnp.maximum(dpo[0] + dpo[1], 1.0)
    din = jnp.maximum(dpi[0] + dpi[1], 1.0)
    no = lax.rsqrt(dout)
    ni = lax.rsqrt(din)
    nin_ref[...] = jnp.broadcast_to((1.0 - ALPHA) * ni, (_BR, HC))
    nout_ref[...] = jnp.broadcast_to(no, (_BR, HC))


def _norm_kernel(dpo, dpi):
    grid = (N // _BR,)
    outm = jax.ShapeDtypeStruct((N, HC), jnp.float32)
    return pl.pallas_call(
        _norm_body,
        grid=grid,
        in_specs=[
            pl.BlockSpec((NC, _BR, 1), lambda i: (0, i, 0)),
            pl.BlockSpec((NC, _BR, 1), lambda i: (0, i, 0)),
        ],
        out_specs=[
            pl.BlockSpec((_BR, HC), lambda i: (i, 0)),
            pl.BlockSpec((_BR, HC), lambda i: (i, 0)),
        ],
        out_shape=[outm, outm],
    )(dpo, dpi)


@functools.partial(
    pl.kernel,
    out_type=[jax.ShapeDtypeStruct((NC * N, HC), jnp.float32),
              jax.ShapeDtypeStruct((N, C), jnp.float32)],
    mesh=_mesh,
    compiler_params=_sc_params,
    scratch_types=[
        pltpu.VMEM((NGB, GB), jnp.int32),
        pltpu.VMEM((NGB, GB), jnp.int32),
        pltpu.VMEM((NBUF, GB, HC), jnp.float32),
        pltpu.VMEM((RCH, HC), jnp.float32),
        pltpu.VMEM_SHARED((N, HC), jnp.float32),
        pltpu.SemaphoreType.DMA((NBUF,)),
        pltpu.SemaphoreType.DMA((NBUF,)),
    ],
)
def _prop_kernel(h0_a, h0_b, ninm, noutm, srcr, dstr, s_buf, out64,
                 src_v, dst_v, rows_v, zch, acc, gsem, ssem):
    accv = rows_v.at[4]
    cid = lax.axis_index("c")
    sid = lax.axis_index("s")
    base_row = sid * RPT

    off = cid * N

    pltpu.sync_copy(srcr.at[cid, sid], src_v)
    pltpu.sync_copy(dstr.at[sid], dst_v)

    def fz(i, _):
        zch[i // (HC // 16), pl.ds((i % (HC // 16)) * 16, 16)] = (
            jnp.zeros((16,), jnp.float32))
        return 0
    lax.fori_loop(0, RCH * (HC // 16), fz, 0)

    def cinit(t, _):
        r0 = base_row + t * RCH

        @pl.when(cid == 0)
        def _():
            pltpu.sync_copy(h0_a.at[pl.ds(r0, RCH)], rows_v.at[1])

        @pl.when(cid == 1)
        def _():
            pltpu.sync_copy(h0_b.at[pl.ds(r0, RCH)], rows_v.at[1])

        pltpu.sync_copy(noutm.at[pl.ds(r0, RCH)], rows_v.at[3])

        def iloop(i, _):
            r = i // (HC // 16)
            sl = pl.ds((i % (HC // 16)) * 16, 16)
            rows_v[1, r, sl] = rows_v[1, r, sl] * rows_v[3, r, sl]
            return 0
        lax.fori_loop(0, RCH * (HC // 16), iloop, 0)

        pltpu.sync_copy(rows_v.at[1], s_buf.at[pl.ds(off + r0, RCH)])
        pltpu.sync_copy(zch, acc.at[pl.ds(r0, RCH)])
        return 0
    lax.fori_loop(0, NCH, cinit, 0)

    plsc.subcore_barrier()

    def step(k, _):
        for p in range(NBUF - 1):
            pltpu.async_copy(s_buf.at[src_v.at[p]], rows_v.at[p],
                             gsem.at[p])

        def body(j, _):
            b = j % NBUF
            ahead = (j + NBUF - 1) % NBUF

            @pl.when(j >= 1)
            def _():
                pltpu.make_async_copy(rows_v.at[ahead],
                                      acc.at[dst_v.at[j - 1]],
                                      ssem.at[ahead]).wait()

            @pl.when(j + NBUF - 1 < NGB)
            def _():
                pltpu.async_copy(s_buf.at[src_v.at[j + NBUF - 1]],
                                 rows_v.at[ahead], gsem.at[ahead])

            pltpu.make_async_copy(s_buf.at[src_v.at[j]], rows_v.at[b],
                                  gsem.at[b]).wait()
            pltpu.async_copy(rows_v.at[b], acc.at[dst_v.at[j]], ssem.at[b],
                             add=True)
            return 0
        lax.fori_loop(0, NGB, body, 0)

        pltpu.make_async_copy(rows_v.at[(NGB - 1) % NBUF],
                              acc.at[dst_v.at[NGB - 1]],
                              ssem.at[(NGB - 1) % NBUF]).wait()

        plsc.subcore_barrier()

        last = k == K_PROP - 1

        def issue_loads(t, g4):
            r0 = base_row + t * RCH

            @pl.when(cid == 0)
            def _():
                pltpu.async_copy(h0_a.at[pl.ds(r0, RCH)],
                                 rows_v.at[g4 + 1], gsem.at[g4 + 1])

            @pl.when(cid == 1)
            def _():
                pltpu.async_copy(h0_b.at[pl.ds(r0, RCH)],
                                 rows_v.at[g4 + 1], gsem.at[g4 + 1])

            pltpu.async_copy(acc.at[pl.ds(r0, RCH)], rows_v.at[g4 + 0],
                             gsem.at[g4 + 0])
            pltpu.async_copy(ninm.at[pl.ds(r0, RCH)],
                             rows_v.at[g4 + 2], gsem.at[g4 + 2])
            pltpu.async_copy(noutm.at[pl.ds(r0, RCH)],
                             rows_v.at[g4 + 3], gsem.at[g4 + 3])

        issue_loads(0, 0)

        def comb(t, _):
            g4 = (t % 2) * 4
            og4 = ((t + 1) % 2) * 4
            r0 = base_row + t * RCH

            @pl.when(jnp.logical_and(t >= 1, jnp.logical_not(last)))
            def _():
                pltpu.make_async_copy(
                    rows_v.at[og4],
                    s_buf.at[pl.ds(off + r0 - RCH, RCH)],
                    ssem.at[og4]).wait()

            @pl.when(jnp.logical_and(t >= 1, last))
            def _():
                pltpu.make_async_copy(
                    rows_v.at[og4],
                    out64.at[pl.ds(r0 - RCH, RCH), pl.ds(cid * HC, HC)],
                    ssem.at[og4]).wait()

            @pl.when(t + 1 < NCH)
            def _():
                issue_loads(t + 1, og4)

            for q in range(4):
                pltpu.make_async_copy(acc.at[pl.ds(r0, RCH)],
                                      rows_v.at[g4 + q],
                                      gsem.at[g4 + q]).wait()

            pltpu.sync_copy(zch, acc.at[pl.ds(r0, RCH)])

            def rowloop(i, _):
                r = i // (HC // 16)
                sl = pl.ds((i % (HC // 16)) * 16, 16)
                no = jnp.where(last, 1.0, rows_v[g4 + 3, r, sl])
                rows_v[g4, r, sl] = (rows_v[g4, r, sl]
                                     * rows_v[g4 + 2, r, sl]
                                     + ALPHA * rows_v[g4 + 1, r, sl]) * no
                return 0
            lax.fori_loop(0, RCH * (HC // 16), rowloop, 0)

            @pl.when(jnp.logical_not(last))
            def _():
                pltpu.async_copy(rows_v.at[g4],
                                 s_buf.at[pl.ds(off + r0, RCH)],
                                 ssem.at[g4])

            @pl.when(last)
            def _():
                pltpu.async_copy(
                    rows_v.at[g4],
                    out64.at[pl.ds(r0, RCH), pl.ds(cid * HC, HC)],
                    ssem.at[g4])
            return 0
        lax.fori_loop(0, NCH, comb, 0)

        g4l = ((NCH - 1) % 2) * 4
        r0l = base_row + (NCH - 1) * RCH

        @pl.when(jnp.logical_not(last))
        def _():
            pltpu.make_async_copy(
                rows_v.at[g4l],
                s_buf.at[pl.ds(off + r0l, RCH)],
                ssem.at[g4l]).wait()

        @pl.when(last)
        def _():
            pltpu.make_async_copy(
                rows_v.at[g4l],
                out64.at[pl.ds(r0l, RCH), pl.ds(cid * HC, HC)],
                ssem.at[g4l]).wait()

        plsc.subcore_barrier()
        return 0
    lax.fori_loop(0, K_PROP, step, 0)


def kernel(features, edge_index, W0, b0, W1, b1, W2, b2):
    src = edge_index[0]
    dst = edge_index[1]

    dpart = _deg_kernel(src.reshape(NW, NB, CE), dst.reshape(NW, NB, CE))
    dpo = dpart[:, 0, :].reshape(NC, N, 1)
    dpi = dpart[:, 1, :].reshape(NC, N, 1)

    h0_a, h0_b = _mlp_kernel(features, W0, b0, W1, b1, W2, b2)
    ninm, noutm = _norm_kernel(dpo, dpi)

    src2 = jnp.stack([src, src + N]).reshape(NC, NS, NGB, GB)
    _, out64 = _prop_kernel(h0_a, h0_b, ninm, noutm,
                            src2, dst.reshape(NS, NGB, GB))
    return out64


# speedup vs baseline: 1.3937x; 1.0008x over previous
import functools

import jax
import jax.numpy as jnp
from jax import lax
from jax.experimental import pallas as pl
from jax.experimental.pallas import tpu as pltpu
from jax.experimental.pallas import tpu_sc as plsc

N = 10000
E = 320000
D = 128
H = 256
C = 64
K_PROP = 10
ALPHA = 0.1

NC = 2
NS = 16
NW = NC * NS
EPW = E // NW
CE = 125
NB = EPW // CE

HC = C // NC
TPE = E // NS
GB = 125
NGB = TPE // GB
NBUF = 10
RPT = N // NS
RCH = 125
NCH = RPT // RCH

_mesh = plsc.VectorSubcoreMesh(core_axis_name="c", subcore_axis_name="s",
                               num_cores=NC, num_subcores=NS)
_sc_params = pltpu.CompilerParams(use_tc_tiling_on_sc=False)


def _worker_id():
    return lax.axis_index("s") * NC + lax.axis_index("c")


@functools.partial(
    pl.kernel,
    out_type=jax.ShapeDtypeStruct((NC, 2, N), jnp.float32),
    mesh=_mesh,
    compiler_params=_sc_params,
    scratch_types=[
        pltpu.VMEM((NB, CE), jnp.int32),
        pltpu.VMEM((NB, CE), jnp.int32),
        pltpu.VMEM((128,), jnp.float32),
        pltpu.VMEM((2000,), jnp.float32),
        pltpu.VMEM_SHARED((N,), jnp.float32),
        pltpu.VMEM_SHARED((N,), jnp.float32),
        pltpu.SemaphoreType.DMA((4,)),
    ],
)
def _deg_kernel(src_hbm, dst_hbm, dpart, src_v, dst_v, ones_v, z_v,
                acc_out, acc_in, dsem):
    cid = lax.axis_index("c")
    sid = lax.axis_index("s")
    wid = _worker_id()

    def fill_ones(i, _):
        ones_v[pl.ds(i * 16, 16)] = jnp.ones((16,), jnp.float32)
        return 0
    lax.fori_loop(0, 128 // 16, fill_ones, 0)

    def fill_z(i, _):
        z_v[pl.ds(i * 16, 16)] = jnp.zeros((16,), jnp.float32)
        return 0
    lax.fori_loop(0, 2000 // 16, fill_z, 0)

    @pl.when(sid < 5)
    def _():
        pltpu.sync_copy(z_v, acc_out.at[pl.ds(sid * 2000, 2000)])
        pltpu.sync_copy(z_v, acc_in.at[pl.ds(sid * 2000, 2000)])

    plsc.subcore_barrier()

    pltpu.sync_copy(src_hbm.at[wid], src_v)
    pltpu.sync_copy(dst_hbm.at[wid], dst_v)

    def body(j, _):
        p2 = (j % 2) * 2

        @pl.when(j >= 2)
        def _():
            pltpu.make_async_copy(ones_v.at[pl.ds(0, CE)],
                                  acc_out.at[src_v.at[j - 2]],
                                  dsem.at[p2]).wait()
            pltpu.make_async_copy(ones_v.at[pl.ds(0, CE)],
                                  acc_in.at[dst_v.at[j - 2]],
                                  dsem.at[p2 + 1]).wait()

        pltpu.async_copy(ones_v.at[pl.ds(0, CE)], acc_out.at[src_v.at[j]],
                         dsem.at[p2], add=True)
        pltpu.async_copy(ones_v.at[pl.ds(0, CE)], acc_in.at[dst_v.at[j]],
                         dsem.at[p2 + 1], add=True)
        return 0
    lax.fori_loop(0, NB, body, 0)

    def dega_drain(d, _):
        j = NB - 2 + d
        p2 = (j % 2) * 2
        pltpu.make_async_copy(ones_v.at[pl.ds(0, CE)],
                              acc_out.at[src_v.at[j]], dsem.at[p2]).wait()
        pltpu.make_async_copy(ones_v.at[pl.ds(0, CE)],
                              acc_in.at[dst_v.at[j]],
                              dsem.at[p2 + 1]).wait()
        return 0
    lax.fori_loop(0, 2, dega_drain, 0)

    plsc.subcore_barrier()

    @pl.when(sid < 10)
    def _():
        pltpu.sync_copy(acc_out.at[pl.ds(sid * 1000, 1000)],
                        dpart.at[cid, 0, pl.ds(sid * 1000, 1000)])
        pltpu.sync_copy(acc_in.at[pl.ds(sid * 1000, 1000)],
                        dpart.at[cid, 1, pl.ds(sid * 1000, 1000)])


_BR = 1000


def _mlp_body(feat, w0, b0, w1, b1, w2, b2, h0a_ref, h0b_ref):
    x = feat[...]
    h = jnp.maximum(jnp.dot(x, w0[...], preferred_element_type=jnp.float32)
                    + b0[...], 0.0)
    h = jnp.maximum(jnp.dot(h, w1[...], preferred_element_type=jnp.float32)
                    + b1[...], 0.0)
    h = jnp.dot(h, w2[...], preferred_element_type=jnp.float32) + b2[...]
    h0a_ref[...] = h[:, :HC]
    h0b_ref[...] = h[:, HC:]


def _mlp_kernel(features, W0, b0, W1, b1, W2, b2):
    grid = (N // _BR,)
    outh = jax.ShapeDtypeStruct((N, HC), jnp.float32)
    return pl.pallas_call(
        _mlp_body,
        grid=grid,
        in_specs=[
            pl.BlockSpec((_BR, D), lambda i: (i, 0)),
            pl.BlockSpec((D, H), lambda i: (0, 0)),
            pl.BlockSpec((1, H), lambda i: (0, 0)),
            pl.BlockSpec((H, H), lambda i: (0, 0)),
            pl.BlockSpec((1, H), lambda i: (0, 0)),
            pl.BlockSpec((H, C), lambda i: (0, 0)),
            pl.BlockSpec((1, C), lambda i: (0, 0)),
        ],
        out_specs=[
            pl.BlockSpec((_BR, HC), lambda i: (i, 0)),
            pl.BlockSpec((_BR, HC), lambda i: (i, 0)),
        ],
        out_shape=[outh, outh],
    )(features, W0, b0.reshape(1, H), W1, b1.reshape(1, H),
      W2, b2.reshape(1, C))


def _norm_body(dpo, dpi, nin_ref, nout_ref):
    dout = jnp.maximum(dpo[0] + dpo[1], 1.0)
    din = jnp.maximum(dpi[0] + dpi[1], 1.0)
    no = lax.rsqrt(dout)
    ni = lax.rsqrt(din)
    nin_ref[...] = jnp.broadcast_to((1.0 - ALPHA) * ni, (_BR, HC))
    nout_ref[...] = jnp.broadcast_to(no, (_BR, HC))


def _norm_kernel(dpo, dpi):
    grid = (N // _BR,)
    outm = jax.ShapeDtypeStruct((N, HC), jnp.float32)
    return pl.pallas_call(
        _norm_body,
        grid=grid,
        in_specs=[
            pl.BlockSpec((NC, _BR, 1), lambda i: (0, i, 0)),
            pl.BlockSpec((NC, _BR, 1), lambda i: (0, i, 0)),
        ],
        out_specs=[
            pl.BlockSpec((_BR, HC), lambda i: (i, 0)),
            pl.BlockSpec((_BR, HC), lambda i: (i, 0)),
        ],
        out_shape=[outm, outm],
    )(dpo, dpi)


@functools.partial(
    pl.kernel,
    out_type=[jax.ShapeDtypeStruct((NC * N, HC), jnp.float32),
              jax.ShapeDtypeStruct((N, C), jnp.float32)],
    mesh=_mesh,
    compiler_params=_sc_params,
    scratch_types=[
        pltpu.VMEM((NGB, GB), jnp.int32),
        pltpu.VMEM((NGB, GB), jnp.int32),
        pltpu.VMEM((NBUF, GB, HC), jnp.float32),
        pltpu.VMEM((RCH, HC), jnp.float32),
        pltpu.VMEM_SHARED((N, HC), jnp.float32),
        pltpu.SemaphoreType.DMA((NBUF,)),
        pltpu.SemaphoreType.DMA((NBUF,)),
    ],
)
def _prop_kernel(h0_a, h0_b, ninm, noutm, srcr, dstr, s_buf, out64,
                 src_v, dst_v, rows_v, zch, acc, gsem, ssem):
    accv = rows_v.at[4]
    cid = lax.axis_index("c")
    sid = lax.axis_index("s")
    base_row = sid * RPT

    off = cid * N

    pltpu.sync_copy(srcr.at[cid, sid], src_v)
    pltpu.sync_copy(dstr.at[sid], dst_v)

    def fz(i, _):
        zch[i // (HC // 16), pl.ds((i % (HC // 16)) * 16, 16)] = (
            jnp.zeros((16,), jnp.float32))
        return 0
    lax.fori_loop(0, RCH * (HC // 16), fz, 0)

    def cinit(t, _):
        r0 = base_row + t * RCH

        @pl.when(cid == 0)
        def _():
            pltpu.sync_copy(h0_a.at[pl.ds(r0, RCH)], rows_v.at[1])

        @pl.when(cid == 1)
        def _():
            pltpu.sync_copy(h0_b.at[pl.ds(r0, RCH)], rows_v.at[1])

        pltpu.sync_copy(noutm.at[pl.ds(r0, RCH)], rows_v.at[3])

        def iloop(i, _):
            r = i // (HC // 16)
            sl = pl.ds((i % (HC // 16)) * 16, 16)
            rows_v[1, r, sl] = rows_v[1, r, sl] * rows_v[3, r, sl]
            return 0
        lax.fori_loop(0, RCH * (HC // 16), iloop, 0)

        pltpu.sync_copy(rows_v.at[1], s_buf.at[pl.ds(off + r0, RCH)])
        pltpu.sync_copy(zch, acc.at[pl.ds(r0, RCH)])
        return 0
    lax.fori_loop(0, NCH, cinit, 0)

    plsc.subcore_barrier()

    def step(k, _):
        for p in range(NBUF - 1):
            pltpu.async_copy(s_buf.at[src_v.at[p]], rows_v.at[p],
                             gsem.at[p])

        def body(j, _):
            b = j % NBUF
            ahead = (j + NBUF - 1) % NBUF

            @pl.when(j >= 1)
            def _():
                pltpu.make_async_copy(rows_v.at[ahead],
                                      acc.at[dst_v.at[j - 1]],
                                      ssem.at[ahead]).wait()

            @pl.when(j + NBUF - 1 < NGB)
            def _():
                pltpu.async_copy(s_buf.at[src_v.at[j + NBUF - 1]],
                                 rows_v.at[ahead], gsem.at[ahead])

            pltpu.make_async_copy(s_buf.at[src_v.at[j]], rows_v.at[b],
                                  gsem.at[b]).wait()
            pltpu.async_copy(rows_v.at[b], acc.at[dst_v.at[j]], ssem.at[b],
                             add=True)
            return 0
        lax.fori_loop(0, NGB, body, 0)

        pltpu.make_async_copy(rows_v.at[(NGB - 1) % NBUF],
                              acc.at[dst_v.at[NGB - 1]],
                              ssem.at[(NGB - 1) % NBUF]).wait()

        plsc.subcore_barrier()

        last = k == K_PROP - 1

        def issue_loads(t, g4):
            r0 = base_row + t * RCH

            @pl.when(cid == 0)
            def _():
                pltpu.async_copy(h0_a.at[pl.ds(r0, RCH)],
                                 rows_v.at[g4 + 1], gsem.at[g4 + 1])

            @pl.when(cid == 1)
            def _():
                pltpu.async_copy(h0_b.at[pl.ds(r0, RCH)],
                                 rows_v.at[g4 + 1], gsem.at[g4 + 1])

            pltpu.async_copy(acc.at[pl.ds(r0, RCH)], rows_v.at[g4 + 0],
                             gsem.at[g4 + 0])
            pltpu.async_copy(ninm.at[pl.ds(r0, RCH)],
                             rows_v.at[g4 + 2], gsem.at[g4 + 2])
            pltpu.async_copy(noutm.at[pl.ds(r0, RCH)],
                             rows_v.at[g4 + 3], gsem.at[g4 + 3])

        issue_loads(0, 0)

        def comb(t, _):
            g4 = (t % 2) * 4
            og4 = ((t + 1) % 2) * 4
            r0 = base_row + t * RCH

            @pl.when(jnp.logical_and(t >= 1, jnp.logical_not(last)))
            def _():
                pltpu.make_async_copy(
                    rows_v.at[og4],
                    s_buf.at[pl.ds(off + r0 - RCH, RCH)],
                    ssem.at[og4]).wait()

            @pl.when(jnp.logical_and(t >= 1, last))
            def _():
                pltpu.make_async_copy(
                    rows_v.at[og4],
                    out64.at[pl.ds(r0 - RCH, RCH), pl.ds(cid * HC, HC)],
                    ssem.at[og4]).wait()

            @pl.when(t + 1 < NCH)
            def _():
                issue_loads(t + 1, og4)

            for q in range(4):
                pltpu.make_async_copy(acc.at[pl.ds(r0, RCH)],
                                      rows_v.at[g4 + q],
                                      gsem.at[g4 + q]).wait()

            pltpu.sync_copy(zch, acc.at[pl.ds(r0, RCH)])

            def rowloop(i, _):
                r = i // (HC // 16)
                sl = pl.ds((i % (HC // 16)) * 16, 16)
                no = jnp.where(last, 1.0, rows_v[g4 + 3, r, sl])
                rows_v[g4, r, sl] = (rows_v[g4, r, sl]
                                     * rows_v[g4 + 2, r, sl]
                                     + ALPHA * rows_v[g4 + 1, r, sl]) * no
                return 0
            lax.fori_loop(0, RCH * (HC // 16), rowloop, 0)

            @pl.when(jnp.logical_not(last))
            def _():
                pltpu.async_copy(rows_v.at[g4],
                                 s_buf.at[pl.ds(off + r0, RCH)],
                                 ssem.at[g4])

            @pl.when(last)
            def _():
                pltpu.async_copy(
                    rows_v.at[g4],
                    out64.at[pl.ds(r0, RCH), pl.ds(cid * HC, HC)],
                    ssem.at[g4])
            return 0
        lax.fori_loop(0, NCH, comb, 0)

        g4l = ((NCH - 1) % 2) * 4
        r0l = base_row + (NCH - 1) * RCH

        @pl.when(jnp.logical_not(last))
        def _():
            pltpu.make_async_copy(
                rows_v.at[g4l],
                s_buf.at[pl.ds(off + r0l, RCH)],
                ssem.at[g4l]).wait()

        @pl.when(last)
        def _():
            pltpu.make_async_copy(
                rows_v.at[g4l],
                out64.at[pl.ds(r0l, RCH), pl.ds(cid * HC, HC)],
                ssem.at[g4l]).wait()

        plsc.subcore_barrier()
        return 0
    lax.fori_loop(0, K_PROP, step, 0)


def kernel(features, edge_index, W0, b0, W1, b1, W2, b2):
    src = edge_index[0]
    dst = edge_index[1]

    dpart = _deg_kernel(src.reshape(NW, NB, CE), dst.reshape(NW, NB, CE))
    dpo = dpart[:, 0, :].reshape(NC, N, 1)
    dpi = dpart[:, 1, :].reshape(NC, N, 1)

    h0_a, h0_b = _mlp_kernel(features, W0, b0, W1, b1, W2, b2)
    ninm, noutm = _norm_kernel(dpo, dpi)

    src2 = jnp.stack([src, src + N]).reshape(NC, NS, NGB, GB)
    _, out64 = _prop_kernel(h0_a, h0_b, ninm, noutm,
                            src2, dst.reshape(NS, NGB, GB))
    return out64


# speedup vs baseline: 1.4156x; 1.0157x over previous
import functools

import jax
import jax.numpy as jnp
from jax import lax
from jax.experimental import pallas as pl
from jax.experimental.pallas import tpu as pltpu
from jax.experimental.pallas import tpu_sc as plsc

N = 10000
E = 320000
D = 128
H = 256
C = 64
K_PROP = 10
ALPHA = 0.1

NC = 2
NS = 16
NW = NC * NS
EPW = E // NW
CE = 125
NB = EPW // CE

HC = C // NC
TPE = E // NS
GB = 125
NGB = TPE // GB
NBUF = 10
RPT = N // NS
RCH = 125
NCH = RPT // RCH

_mesh = plsc.VectorSubcoreMesh(core_axis_name="c", subcore_axis_name="s",
                               num_cores=NC, num_subcores=NS)
_sc_params = pltpu.CompilerParams(use_tc_tiling_on_sc=False)


def _worker_id():
    return lax.axis_index("s") * NC + lax.axis_index("c")


@functools.partial(
    pl.kernel,
    out_type=jax.ShapeDtypeStruct((NC, 2, N), jnp.float32),
    mesh=_mesh,
    compiler_params=_sc_params,
    scratch_types=[
        pltpu.VMEM((NB, CE), jnp.int32),
        pltpu.VMEM((NB, CE), jnp.int32),
        pltpu.VMEM((128,), jnp.float32),
        pltpu.VMEM((2000,), jnp.float32),
        pltpu.VMEM_SHARED((N,), jnp.float32),
        pltpu.VMEM_SHARED((N,), jnp.float32),
        pltpu.SemaphoreType.DMA((4,)),
    ],
)
def _deg_kernel(src_hbm, dst_hbm, dpart, src_v, dst_v, ones_v, z_v,
                acc_out, acc_in, dsem):
    cid = lax.axis_index("c")
    sid = lax.axis_index("s")
    wid = _worker_id()

    def fill_ones(i, _):
        ones_v[pl.ds(i * 16, 16)] = jnp.ones((16,), jnp.float32)
        return 0
    lax.fori_loop(0, 128 // 16, fill_ones, 0)

    def fill_z(i, _):
        z_v[pl.ds(i * 16, 16)] = jnp.zeros((16,), jnp.float32)
        return 0
    lax.fori_loop(0, 2000 // 16, fill_z, 0)

    @pl.when(sid < 5)
    def _():
        pltpu.sync_copy(z_v, acc_out.at[pl.ds(sid * 2000, 2000)])
        pltpu.sync_copy(z_v, acc_in.at[pl.ds(sid * 2000, 2000)])

    plsc.subcore_barrier()

    pltpu.sync_copy(src_hbm.at[wid], src_v)
    pltpu.sync_copy(dst_hbm.at[wid], dst_v)

    def body(j, _):
        p2 = (j % 2) * 2

        @pl.when(j >= 2)
        def _():
            pltpu.make_async_copy(ones_v.at[pl.ds(0, CE)],
                                  acc_out.at[src_v.at[j - 2]],
                                  dsem.at[p2]).wait()
            pltpu.make_async_copy(ones_v.at[pl.ds(0, CE)],
                                  acc_in.at[dst_v.at[j - 2]],
                                  dsem.at[p2 + 1]).wait()

        pltpu.async_copy(ones_v.at[pl.ds(0, CE)], acc_out.at[src_v.at[j]],
                         dsem.at[p2], add=True)
        pltpu.async_copy(ones_v.at[pl.ds(0, CE)], acc_in.at[dst_v.at[j]],
                         dsem.at[p2 + 1], add=True)
        return 0
    lax.fori_loop(0, NB, body, 0)

    def dega_drain(d, _):
        j = NB - 2 + d
        p2 = (j % 2) * 2
        pltpu.make_async_copy(ones_v.at[pl.ds(0, CE)],
                              acc_out.at[src_v.at[j]], dsem.at[p2]).wait()
        pltpu.make_async_copy(ones_v.at[pl.ds(0, CE)],
                              acc_in.at[dst_v.at[j]],
                              dsem.at[p2 + 1]).wait()
        return 0
    lax.fori_loop(0, 2, dega_drain, 0)

    plsc.subcore_barrier()

    @pl.when(sid < 10)
    def _():
        pltpu.sync_copy(acc_out.at[pl.ds(sid * 1000, 1000)],
                        dpart.at[cid, 0, pl.ds(sid * 1000, 1000)])
        pltpu.sync_copy(acc_in.at[pl.ds(sid * 1000, 1000)],
                        dpart.at[cid, 1, pl.ds(sid * 1000, 1000)])


_BR = 1000


def _mlp_body(feat, w0, b0, w1, b1, w2, b2, h0a_ref, h0b_ref):
    x = feat[...]
    h = jnp.maximum(jnp.dot(x, w0[...], preferred_element_type=jnp.float32)
                    + b0[...], 0.0)
    h = jnp.maximum(jnp.dot(h, w1[...], preferred_element_type=jnp.float32)
                    + b1[...], 0.0)
    h = jnp.dot(h, w2[...], preferred_element_type=jnp.float32) + b2[...]
    h0a_ref[...] = h[:, :HC]
    h0b_ref[...] = h[:, HC:]


def _mlp_kernel(features, W0, b0, W1, b1, W2, b2):
    grid = (N // _BR,)
    outh = jax.ShapeDtypeStruct((N, HC), jnp.float32)
    return pl.pallas_call(
        _mlp_body,
        grid=grid,
        in_specs=[
            pl.BlockSpec((_BR, D), lambda i: (i, 0)),
            pl.BlockSpec((D, H), lambda i: (0, 0)),
            pl.BlockSpec((1, H), lambda i: (0, 0)),
            pl.BlockSpec((H, H), lambda i: (0, 0)),
            pl.BlockSpec((1, H), lambda i: (0, 0)),
            pl.BlockSpec((H, C), lambda i: (0, 0)),
            pl.BlockSpec((1, C), lambda i: (0, 0)),
        ],
        out_specs=[
            pl.BlockSpec((_BR, HC), lambda i: (i, 0)),
            pl.BlockSpec((_BR, HC), lambda i: (i, 0)),
        ],
        out_shape=[outh, outh],
    )(features, W0, b0.reshape(1, H), W1, b1.reshape(1, H),
      W2, b2.reshape(1, C))


def _norm_body(dpo, dpi, nin_ref, nout_ref):
    dout = jnp.maximum(dpo[0] + dpo[1], 1.0)
    din = jnp.maximum(dpi[0] + dpi[1], 1.0)
    no = lax.rsqrt(dout)
    ni = lax.rsqrt(din)
    nin_ref[...] = jnp.broadcast_to((1.0 - ALPHA) * ni, (_BR, HC))
    nout_ref[...] = jnp.broadcast_to(no, (_BR, HC))


def _norm_kernel(dpo, dpi):
    grid = (N // _BR,)
    outm = jax.ShapeDtypeStruct((N, HC), jnp.float32)
    return pl.pallas_call(
        _norm_body,
        grid=grid,
        in_specs=[
            pl.BlockSpec((NC, _BR, 1), lambda i: (0, i, 0)),
            pl.BlockSpec((NC, _BR, 1), lambda i: (0, i, 0)),
        ],
        out_specs=[
            pl.BlockSpec((_BR, HC), lambda i: (i, 0)),
            pl.BlockSpec((_BR, HC), lambda i: (i, 0)),
        ],
        out_shape=[outm, outm],
    )(dpo, dpi)


@functools.partial(
    pl.kernel,
    out_type=[jax.ShapeDtypeStruct((NC * N, HC), jnp.float32),
              jax.ShapeDtypeStruct((N, C), jnp.float32)],
    mesh=_mesh,
    compiler_params=_sc_params,
    scratch_types=[
        pltpu.VMEM((NGB, GB), jnp.int32),
        pltpu.VMEM((NGB, GB), jnp.int32),
        pltpu.VMEM((NBUF, GB, HC), jnp.float32),
        pltpu.VMEM((RCH, HC), jnp.float32),
        pltpu.VMEM_SHARED((N, HC), jnp.float32),
        pltpu.SemaphoreType.DMA((NBUF,)),
        pltpu.SemaphoreType.DMA((NBUF,)),
    ],
)
def _prop_kernel(h0_a, h0_b, ninm, noutm, srcr, dstr, s_buf, out64,
                 src_v, dst_v, rows_v, zch, acc, gsem, ssem):
    accv = rows_v.at[4]
    cid = lax.axis_index("c")
    sid = lax.axis_index("s")
    base_row = sid * RPT

    off = cid * N

    pltpu.sync_copy(srcr.at[cid, sid], src_v)
    pltpu.sync_copy(dstr.at[sid], dst_v)

    def fz(i, _):
        zch[i // (HC // 16), pl.ds((i % (HC // 16)) * 16, 16)] = (
            jnp.zeros((16,), jnp.float32))
        return 0
    lax.fori_loop(0, RCH * (HC // 16), fz, 0)

    def cinit(t, _):
        r0 = base_row + t * RCH

        @pl.when(cid == 0)
        def _():
            pltpu.sync_copy(h0_a.at[pl.ds(r0, RCH)], rows_v.at[1])

        @pl.when(cid == 1)
        def _():
            pltpu.sync_copy(h0_b.at[pl.ds(r0, RCH)], rows_v.at[1])

        pltpu.sync_copy(noutm.at[pl.ds(r0, RCH)], rows_v.at[3])

        def iloop(i, _):
            r = i // (HC // 16)
            sl = pl.ds((i % (HC // 16)) * 16, 16)
            rows_v[1, r, sl] = rows_v[1, r, sl] * rows_v[3, r, sl]
            return 0
        lax.fori_loop(0, RCH * (HC // 16), iloop, 0)

        pltpu.sync_copy(rows_v.at[1], s_buf.at[pl.ds(off + r0, RCH)])
        pltpu.sync_copy(zch, acc.at[pl.ds(r0, RCH)])
        return 0
    lax.fori_loop(0, NCH, cinit, 0)

    plsc.subcore_barrier()

    def step(k, _):
        for p in range(NBUF - 1):
            pltpu.async_copy(s_buf.at[src_v.at[p]], rows_v.at[p],
                             gsem.at[p])

        def body(j, _):
            b = j % NBUF
            ahead = (j + NBUF - 1) % NBUF

            @pl.when(j >= 1)
            def _():
                pltpu.make_async_copy(rows_v.at[ahead],
                                      acc.at[dst_v.at[j - 1]],
                                      ssem.at[ahead]).wait()

            @pl.when(j + NBUF - 1 < NGB)
            def _():
                pltpu.async_copy(s_buf.at[src_v.at[j + NBUF - 1]],
                                 rows_v.at[ahead], gsem.at[ahead])

            pltpu.make_async_copy(s_buf.at[src_v.at[j]], rows_v.at[b],
                                  gsem.at[b]).wait()
            pltpu.async_copy(rows_v.at[b], acc.at[dst_v.at[j]], ssem.at[b],
                             add=True)
            return 0
        lax.fori_loop(0, NGB, body, 0)

        pltpu.make_async_copy(rows_v.at[(NGB - 1) % NBUF],
                              acc.at[dst_v.at[NGB - 1]],
                              ssem.at[(NGB - 1) % NBUF]).wait()

        plsc.subcore_barrier()

        last = k == K_PROP - 1

        def issue_loads(t, g4):
            r0 = base_row + t * RCH

            @pl.when(cid == 0)
            def _():
                pltpu.async_copy(h0_a.at[pl.ds(r0, RCH)],
                                 rows_v.at[g4 + 1], gsem.at[g4 + 1])

            @pl.when(cid == 1)
            def _():
                pltpu.async_copy(h0_b.at[pl.ds(r0, RCH)],
                                 rows_v.at[g4 + 1], gsem.at[g4 + 1])

            pltpu.async_copy(acc.at[pl.ds(r0, RCH)], rows_v.at[g4 + 0],
                             gsem.at[g4 + 0])
            pltpu.async_copy(ninm.at[pl.ds(r0, RCH)],
                             rows_v.at[g4 + 2], gsem.at[g4 + 2])
            pltpu.async_copy(noutm.at[pl.ds(r0, RCH)],
                             rows_v.at[g4 + 3], gsem.at[g4 + 3])

        issue_loads(0, 0)

        def comb(t, _):
            g4 = (t % 2) * 4
            og4 = ((t + 1) % 2) * 4
            r0 = base_row + t * RCH

            @pl.when(jnp.logical_and(t >= 1, jnp.logical_not(last)))
            def _():
                pltpu.make_async_copy(
                    rows_v.at[og4],
                    s_buf.at[pl.ds(off + r0 - RCH, RCH)],
                    ssem.at[og4]).wait()

            @pl.when(jnp.logical_and(t >= 1, last))
            def _():
                pltpu.make_async_copy(
                    rows_v.at[og4],
                    out64.at[pl.ds(r0 - RCH, RCH), pl.ds(cid * HC, HC)],
                    ssem.at[og4]).wait()

            @pl.when(t + 1 < NCH)
            def _():
                issue_loads(t + 1, og4)

            for q in range(4):
                pltpu.make_async_copy(acc.at[pl.ds(r0, RCH)],
                                      rows_v.at[g4 + q],
                                      gsem.at[g4 + q]).wait()

            @pl.when(t >= 2)
            def _():
                pltpu.make_async_copy(zch,
                                      acc.at[pl.ds(r0 - 2 * RCH, RCH)],
                                      ssem.at[8 + t % 2]).wait()
            pltpu.async_copy(zch, acc.at[pl.ds(r0, RCH)],
                             ssem.at[8 + t % 2])

            def rowloop(i, _):
                r = i // (HC // 16)
                sl = pl.ds((i % (HC // 16)) * 16, 16)
                no = jnp.where(last, 1.0, rows_v[g4 + 3, r, sl])
                rows_v[g4, r, sl] = (rows_v[g4, r, sl]
                                     * rows_v[g4 + 2, r, sl]
                                     + ALPHA * rows_v[g4 + 1, r, sl]) * no
                return 0
            lax.fori_loop(0, RCH * (HC // 16), rowloop, 0)

            @pl.when(jnp.logical_not(last))
            def _():
                pltpu.async_copy(rows_v.at[g4],
                                 s_buf.at[pl.ds(off + r0, RCH)],
                                 ssem.at[g4])

            @pl.when(last)
            def _():
                pltpu.async_copy(
                    rows_v.at[g4],
                    out64.at[pl.ds(r0, RCH), pl.ds(cid * HC, HC)],
                    ssem.at[g4])
            return 0
        lax.fori_loop(0, NCH, comb, 0)

        def zdrain(d, _):
            t = NCH - 2 + d
            pltpu.make_async_copy(
                zch, acc.at[pl.ds(base_row + t * RCH, RCH)],
                ssem.at[8 + t % 2]).wait()
            return 0
        lax.fori_loop(0, 2, zdrain, 0)

        g4l = ((NCH - 1) % 2) * 4
        r0l = base_row + (NCH - 1) * RCH

        @pl.when(jnp.logical_not(last))
        def _():
            pltpu.make_async_copy(
                rows_v.at[g4l],
                s_buf.at[pl.ds(off + r0l, RCH)],
                ssem.at[g4l]).wait()

        @pl.when(last)
        def _():
            pltpu.make_async_copy(
                rows_v.at[g4l],
                out64.at[pl.ds(r0l, RCH), pl.ds(cid * HC, HC)],
                ssem.at[g4l]).wait()

        plsc.subcore_barrier()
        return 0
    lax.fori_loop(0, K_PROP, step, 0)


def kernel(features, edge_index, W0, b0, W1, b1, W2, b2):
    src = edge_index[0]
    dst = edge_index[1]

    dpart = _deg_kernel(src.reshape(NW, NB, CE), dst.reshape(NW, NB, CE))
    dpo = dpart[:, 0, :].reshape(NC, N, 1)
    dpi = dpart[:, 1, :].reshape(NC, N, 1)

    h0_a, h0_b = _mlp_kernel(features, W0, b0, W1, b1, W2, b2)
    ninm, noutm = _norm_kernel(dpo, dpi)

    src2 = jnp.stack([src, src + N]).reshape(NC, NS, NGB, GB)
    _, out64 = _prop_kernel(h0_a, h0_b, ninm, noutm,
                            src2, dst.reshape(NS, NGB, GB))
    return out64
